# baseline (speedup 1.0000x reference)
import functools

import jax
import jax.numpy as jnp
from jax import lax
from jax.experimental import pallas as pl
from jax.experimental.pallas import tpu as pltpu
from jax.experimental.pallas import tpu_sc as plsc

NUM_CORES = 2
LANES = 16
NUM_WORKERS = 32

BATCH = 16384
DIM = 32
B_PER_W = BATCH // NUM_WORKERS
GRP = 16
NGRP = B_PER_W // GRP

_mesh = plsc.VectorSubcoreMesh(core_axis_name="c", subcore_axis_name="s")


@functools.partial(
    pl.kernel,
    out_type=jax.ShapeDtypeStruct((BATCH,), jnp.float32),
    mesh=_mesh,
    scratch_types=[
        pltpu.VMEM((B_PER_W,), jnp.int32),
        pltpu.VMEM((B_PER_W,), jnp.int32),
        pltpu.VMEM((DIM, GRP * 128), jnp.float32),
        pltpu.VMEM((DIM, GRP), jnp.float32),
        pltpu.VMEM((B_PER_W,), jnp.float32),
        pltpu.SemaphoreType.DMA,
    ],
    compiler_params=pltpu.CompilerParams(
        needs_layout_passes=False),
)
def _sc_dot_kernel(utT, stT, uid_hbm, sid_hbm, out_hbm,
                   uids_v, sids_v, blk, ucols, out_v, sem):
    wid = lax.axis_index("s") * NUM_CORES + lax.axis_index("c")
    base = wid * B_PER_W

    pltpu.sync_copy(uid_hbm.at[pl.ds(base, B_PER_W)], uids_v)
    pltpu.sync_copy(sid_hbm.at[pl.ds(base, B_PER_W)], sids_v)

    lane = lax.iota(jnp.int32, LANES)

    def fetch_group(tab, ids_ref, g):
        gv = (ids_ref[pl.ds(g * GRP, GRP)] >> 7) * 128
        for i in range(GRP):
            jblk = pl.multiple_of(gv[i], 128)
            pltpu.async_copy(
                tab.at[:, pl.ds(jblk, 128)],
                blk.at[:, pl.ds(i * 128, 128)], sem)
        pltpu.make_async_copy(tab.at[:, pl.ds(0, GRP * 128)], blk, sem).wait()

    def extract_to(dst_ref, ids_v, g):
        idv = plsc.load_gather(ids_v, [g * GRP + lane])
        col = lane * 128 + (idv & 127)
        for d in range(DIM):
            dst_ref[d, pl.ds(0, GRP)] = plsc.load_gather(
                blk, [jnp.full((LANES,), d, jnp.int32), col])

    def group(g, carry):
        fetch_group(utT, uids_v, g)
        extract_to(ucols, uids_v, g)
        fetch_group(stT, sids_v, g)
        idv = plsc.load_gather(sids_v, [g * GRP + lane])
        col = lane * 128 + (idv & 127)
        acc = jnp.zeros((LANES,), jnp.float32)
        for d in range(DIM):
            sd = plsc.load_gather(
                blk, [jnp.full((LANES,), d, jnp.int32), col])
            acc = acc + sd * ucols[d, pl.ds(0, GRP)]
        out_v[pl.ds(g * GRP, GRP)] = acc
        return carry

    lax.fori_loop(0, NGRP, group, 0)

    pltpu.sync_copy(out_v, out_hbm.at[pl.ds(base, B_PER_W)])


def kernel(user_table, song_table, user_ids, song_ids):
    return _sc_dot_kernel(
        user_table.T, song_table.T,
        user_ids.astype(jnp.int32), song_ids.astype(jnp.int32))
